# Initial kernel scaffold; baseline (speedup 1.0000x reference)
import jax
import jax.numpy as jnp
from jax.experimental import pallas as pl


def kernel(x, edge_index, W1, b1, W2, b2, W3, b3):
    raise NotImplementedError("write your pallas kernel here")


# speedup vs baseline: 2.9825x; 2.9825x over previous
import functools

import jax
import jax.numpy as jnp
from jax import lax
from jax.experimental import pallas as pl
from jax.experimental.pallas import tpu as pltpu
from jax.experimental.pallas import tpu_sc as plsc

N = 10000
E = 320000
D = 128
NC = 2
NS = 16
NW = NC * NS
CHUNK = 128
N_PAD = 10240
E_PAD = NW * 80 * CHUNK
NCH = E_PAD // (NW * CHUNK)
RPT = N_PAD // NS
PAD_IDX = N_PAD - 1


def _agg_body(refs):
    (g_hbm, idx_hbm, zeros_hbm,
     s_out,
     idxv, rows0, rows1, acc, sem0, sem1, isem) = refs

    c = lax.axis_index("c")
    s = lax.axis_index("s")
    wid = c * NS + s

    pltpu.sync_copy(zeros_hbm.at[pl.ds(s * RPT, RPT)],
                    acc.at[pl.ds(s * RPT, RPT)])
    plsc.subcore_barrier()

    rows = (rows0, rows1)
    sems = (sem0, sem1)

    ibase = wid * NCH * 2
    pltpu.sync_copy(idx_hbm.at[pl.ds(ibase, 2)], idxv.at[0])
    pltpu.async_copy(idx_hbm.at[pl.ds(ibase + 2, 2)], idxv.at[1], isem)
    pltpu.async_copy(g_hbm.at[idxv.at[0, 0]], rows0, sem0)

    @pl.loop(0, NCH, step=2)
    def _(jj):
        for b in (0, 1):
            j = jj + b
            pltpu.make_async_copy(g_hbm.at[pl.ds(0, CHUNK)], rows[b],
                                  sems[b]).wait()
            nxt = j + 1

            @pl.when(nxt < NCH)
            def _():
                slot = lax.rem(nxt, 4)
                pltpu.make_async_copy(idx_hbm.at[pl.ds(0, 2)], idxv.at[slot],
                                      isem).wait()
                pltpu.async_copy(g_hbm.at[idxv.at[slot, 0]], rows[1 - b],
                                 sems[1 - b])

            @pl.when(j + 2 < NCH)
            def _():
                slot2 = lax.rem(j + 2, 4)
                pltpu.async_copy(idx_hbm.at[pl.ds(ibase + (j + 2) * 2, 2)],
                                 idxv.at[slot2], isem)

            dslot = lax.rem(j, 4)
            pltpu.sync_copy(rows[b], acc.at[idxv.at[dslot, 1]], add=True)

    plsc.subcore_barrier()
    pltpu.sync_copy(acc.at[pl.ds(s * RPT, RPT)],
                    s_out.at[pl.ds(c * N_PAD + s * RPT, RPT)])


def _deg_body(refs):
    (idx_hbm, zeros_hbm, ones_hbm,
     deg_out,
     idxv, onesv, accd, isem) = refs

    c = lax.axis_index("c")
    s = lax.axis_index("s")
    wid = c * NS + s

    pltpu.sync_copy(zeros_hbm.at[pl.ds(s * RPT, RPT)],
                    accd.at[pl.ds(s * RPT, RPT)])
    for k in range(CHUNK // 8):
        pltpu.sync_copy(ones_hbm, onesv.at[pl.ds(k * 8, 8)])
    plsc.subcore_barrier()

    ibase = wid * NCH * 2
    pltpu.async_copy(idx_hbm.at[pl.ds(ibase, 2)], idxv.at[0], isem)
    pltpu.async_copy(idx_hbm.at[pl.ds(ibase + 2, 2)], idxv.at[1], isem)

    @pl.loop(0, NCH)
    def _(j):
        pltpu.make_async_copy(idx_hbm.at[pl.ds(0, 2)],
                              idxv.at[lax.rem(j, 4)], isem).wait()

        @pl.when(j + 2 < NCH)
        def _():
            slot2 = lax.rem(j + 2, 4)
            pltpu.async_copy(idx_hbm.at[pl.ds(ibase + (j + 2) * 2, 2)],
                             idxv.at[slot2], isem)

        pltpu.sync_copy(onesv, accd.at[idxv.at[lax.rem(j, 4), 1]], add=True)

    plsc.subcore_barrier()
    pltpu.sync_copy(accd.at[pl.ds(s * RPT, RPT)],
                    deg_out.at[pl.ds(c * N_PAD + s * RPT, RPT)])


def _sc_mesh():
    return plsc.VectorSubcoreMesh(core_axis_name="c", subcore_axis_name="s",
                                  num_cores=NC, num_subcores=NS)


@functools.lru_cache(maxsize=None)
def _make_agg():
    scratch = [
        pltpu.VMEM((4, 2, CHUNK), jnp.int32),
        pltpu.VMEM((CHUNK, D), jnp.float32),
        pltpu.VMEM((CHUNK, D), jnp.float32),
        pltpu.VMEM_SHARED((N_PAD, D), jnp.float32),
        pltpu.SemaphoreType.DMA, pltpu.SemaphoreType.DMA,
        pltpu.SemaphoreType.DMA,
    ]
    return pl.kernel(
        lambda *refs: _agg_body(refs),
        out_type=jax.ShapeDtypeStruct((NC * N_PAD, D), jnp.float32),
        mesh=_sc_mesh(),
        scratch_types=scratch,
        name="gin_agg",
    )


@functools.lru_cache(maxsize=None)
def _make_deg():
    scratch = [
        pltpu.VMEM((4, 2, CHUNK), jnp.int32),
        pltpu.VMEM((CHUNK, D), jnp.float32),
        pltpu.VMEM_SHARED((N_PAD, D), jnp.float32),
        pltpu.SemaphoreType.DMA,
    ]
    return pl.kernel(
        lambda *refs: _deg_body(refs),
        out_type=jax.ShapeDtypeStruct((NC * N_PAD, D), jnp.float32),
        mesh=_sc_mesh(),
        scratch_types=scratch,
        name="gin_deg",
    )


_USE_SC_AGG = True
_USE_SC_DEG = True


def _xla_agg(g, idx2, zeros):
    idx4 = idx2.reshape(NW, NCH, 2, CHUNK)
    src = idx4[:, :, 0, :].reshape(-1)
    dst = idx4[:, :, 1, :].reshape(-1)
    half = src.shape[0] // 2
    parts = []
    for c in range(2):
        sel = slice(c * half, (c + 1) * half)
        msg = jnp.take(g, src[sel], axis=0)
        parts.append(jax.ops.segment_sum(msg, dst[sel], num_segments=N_PAD))
    return jnp.concatenate(parts, axis=0)


def _xla_deg(idx2, zeros, ones8):
    idx4 = idx2.reshape(NW, NCH, 2, CHUNK)
    dst = idx4[:, :, 1, :].reshape(-1)
    half = dst.shape[0] // 2
    parts = []
    for c in range(2):
        sel = slice(c * half, (c + 1) * half)
        parts.append(jax.ops.segment_sum(
            jnp.ones((half, D), jnp.float32), dst[sel], num_segments=N_PAD))
    return jnp.concatenate(parts, axis=0)


def _agg(*args):
    if _USE_SC_AGG:
        return _make_agg()(*args)
    return _xla_agg(*args)


def _deg(*args):
    if _USE_SC_DEG:
        return _make_deg()(*args)
    return _xla_deg(*args)


_BLK = 1024
_GRID = N_PAD // _BLK


def _mm_body(x_ref, w_ref, o_ref):
    o_ref[...] = lax.dot_general(
        x_ref[...], w_ref[...], (((1,), (1,)), ((), ())),
        preferred_element_type=jnp.float32)


def _mm1(x, W):
    return pl.pallas_call(
        _mm_body,
        grid=(_GRID,),
        in_specs=[
            pl.BlockSpec((_BLK, D), lambda i: (i, 0)),
            pl.BlockSpec((D, D), lambda i: (0, 0)),
        ],
        out_specs=pl.BlockSpec((_BLK, D), lambda i: (i, 0)),
        out_shape=jax.ShapeDtypeStruct((N_PAD, D), jnp.float32),
    )(x, W)


def _combine_mm_body(g_ref, sa_ref, sb_ref, da_ref, db_ref, b_ref, w_ref,
                     o_ref):
    deg = da_ref[:, :1] + db_ref[:, :1]
    dinv = 1.0 / jnp.maximum(deg, 1.0)
    h = g_ref[...] + b_ref[...] + dinv * (sa_ref[...] + sb_ref[...])
    h = jnp.maximum(h, 0.0)
    o_ref[...] = lax.dot_general(
        h, w_ref[...], (((1,), (1,)), ((), ())),
        preferred_element_type=jnp.float32)


def _combine_mm(g, sa, sb, da, db, b, W):
    return pl.pallas_call(
        _combine_mm_body,
        grid=(_GRID,),
        in_specs=[
            pl.BlockSpec((_BLK, D), lambda i: (i, 0)),
            pl.BlockSpec((_BLK, D), lambda i: (i, 0)),
            pl.BlockSpec((_BLK, D), lambda i: (i, 0)),
            pl.BlockSpec((_BLK, D), lambda i: (i, 0)),
            pl.BlockSpec((_BLK, D), lambda i: (i, 0)),
            pl.BlockSpec((1, D), lambda i: (0, 0)),
            pl.BlockSpec((D, D), lambda i: (0, 0)),
        ],
        out_specs=pl.BlockSpec((_BLK, D), lambda i: (i, 0)),
        out_shape=jax.ShapeDtypeStruct((N_PAD, D), jnp.float32),
    )(g, sa, sb, da, db, b, W)


_OBLK = 1000


def _combine_out_body(g_ref, sa_ref, sb_ref, da_ref, db_ref, b_ref, o_ref):
    deg = da_ref[:, :1] + db_ref[:, :1]
    dinv = 1.0 / jnp.maximum(deg, 1.0)
    o_ref[...] = g_ref[...] + b_ref[...] + dinv * (sa_ref[...] + sb_ref[...])


def _combine_out(g, sa, sb, da, db, b):
    return pl.pallas_call(
        _combine_out_body,
        grid=(N // _OBLK,),
        in_specs=[
            pl.BlockSpec((_OBLK, D), lambda i: (i, 0)),
            pl.BlockSpec((_OBLK, D), lambda i: (i, 0)),
            pl.BlockSpec((_OBLK, D), lambda i: (i, 0)),
            pl.BlockSpec((_OBLK, D), lambda i: (i, 0)),
            pl.BlockSpec((_OBLK, D), lambda i: (i, 0)),
            pl.BlockSpec((1, D), lambda i: (0, 0)),
        ],
        out_specs=pl.BlockSpec((_OBLK, D), lambda i: (i, 0)),
        out_shape=jax.ShapeDtypeStruct((N, D), jnp.float32),
    )(g, sa, sb, da, db, b)


def kernel(x, edge_index, W1, b1, W2, b2, W3, b3):
    src = edge_index[0].astype(jnp.int32)
    dst = edge_index[1].astype(jnp.int32)
    pad = jnp.full((E_PAD - E,), PAD_IDX, jnp.int32)
    src3 = jnp.concatenate([src, pad]).reshape(NW, NCH, 1, CHUNK)
    dst3 = jnp.concatenate([dst, pad]).reshape(NW, NCH, 1, CHUNK)
    idx2 = jnp.concatenate([src3, dst3], axis=2).reshape(NW * NCH * 2, CHUNK)
    zeros = jnp.zeros((N_PAD, D), jnp.float32)
    ones8 = jnp.ones((8, D), jnp.float32)
    b1r = b1.reshape(1, D)
    b2r = b2.reshape(1, D)
    b3r = b3.reshape(1, D)

    g1 = _mm1(x, W1)
    deg = _deg(idx2, zeros, ones8)
    s1 = _agg(g1, idx2, zeros)
    da, db = deg[:N_PAD], deg[N_PAD:]
    g2 = _combine_mm(g1, s1[:N_PAD], s1[N_PAD:], da, db, b1r, W2)
    s2 = _agg(g2, idx2, zeros)
    g3 = _combine_mm(g2, s2[:N_PAD], s2[N_PAD:], da, db, b2r, W3)
    s3 = _agg(g3, idx2, zeros)
    return _combine_out(g3, s3[:N_PAD], s3[N_PAD:], da, db, b3r)


# speedup vs baseline: 9.0994x; 3.0510x over previous
import functools

import jax
import jax.numpy as jnp
from jax import lax
from jax.experimental import pallas as pl
from jax.experimental.pallas import tpu as pltpu
from jax.experimental.pallas import tpu_sc as plsc

N = 10000
E = 320000
D = 128
NC = 2
NS = 16
NW = NC * NS
CHUNK = 128
N_PAD = 10240
E_PAD = NW * 80 * CHUNK
NCH = E_PAD // (NW * CHUNK)
RPT = N_PAD // NS
PAD_IDX = N_PAD - 1


def _agg_body(refs):
    (g_hbm, idx_hbm, zeros_hbm,
     s_out,
     idxv, rows0, rows1, acc, sem0, sem1, isem) = refs

    c = lax.axis_index("c")
    s = lax.axis_index("s")
    wid = c * NS + s

    pltpu.sync_copy(zeros_hbm.at[pl.ds(s * RPT, RPT)],
                    acc.at[pl.ds(s * RPT, RPT)])
    plsc.subcore_barrier()

    rows = (rows0, rows1)
    sems = (sem0, sem1)

    ibase = wid * NCH * 2
    pltpu.sync_copy(idx_hbm.at[pl.ds(ibase, 2)], idxv.at[0])
    pltpu.async_copy(idx_hbm.at[pl.ds(ibase + 2, 2)], idxv.at[1], isem)
    pltpu.async_copy(g_hbm.at[idxv.at[0, 0]], rows0, sem0)

    @pl.loop(0, NCH, step=2)
    def _(jj):
        for b in (0, 1):
            j = jj + b
            pltpu.make_async_copy(g_hbm.at[pl.ds(0, CHUNK)], rows[b],
                                  sems[b]).wait()
            nxt = j + 1

            @pl.when(nxt < NCH)
            def _():
                slot = lax.rem(nxt, 4)
                pltpu.make_async_copy(idx_hbm.at[pl.ds(0, 2)], idxv.at[slot],
                                      isem).wait()
                pltpu.async_copy(g_hbm.at[idxv.at[slot, 0]], rows[1 - b],
                                 sems[1 - b])

            @pl.when(j + 2 < NCH)
            def _():
                slot2 = lax.rem(j + 2, 4)
                pltpu.async_copy(idx_hbm.at[pl.ds(ibase + (j + 2) * 2, 2)],
                                 idxv.at[slot2], isem)

            dslot = lax.rem(j, 4)
            pltpu.sync_copy(rows[b], acc.at[idxv.at[dslot, 1]], add=True)

    plsc.subcore_barrier()
    pltpu.sync_copy(acc.at[pl.ds(s * RPT, RPT)],
                    s_out.at[pl.ds(c * N_PAD + s * RPT, RPT)])


def _deg_body(refs):
    (idx_hbm, zeros_hbm, ones_hbm,
     deg_out,
     idxv, onesv, accd, isem) = refs

    c = lax.axis_index("c")
    s = lax.axis_index("s")
    wid = c * NS + s

    pltpu.sync_copy(zeros_hbm.at[pl.ds(s * RPT, RPT)],
                    accd.at[pl.ds(s * RPT, RPT)])
    for k in range(CHUNK // 8):
        pltpu.sync_copy(ones_hbm, onesv.at[pl.ds(k * 8, 8)])
    plsc.subcore_barrier()

    ibase = wid * NCH * 2
    pltpu.async_copy(idx_hbm.at[pl.ds(ibase, 2)], idxv.at[0], isem)
    pltpu.async_copy(idx_hbm.at[pl.ds(ibase + 2, 2)], idxv.at[1], isem)

    @pl.loop(0, NCH)
    def _(j):
        pltpu.make_async_copy(idx_hbm.at[pl.ds(0, 2)],
                              idxv.at[lax.rem(j, 4)], isem).wait()

        @pl.when(j + 2 < NCH)
        def _():
            slot2 = lax.rem(j + 2, 4)
            pltpu.async_copy(idx_hbm.at[pl.ds(ibase + (j + 2) * 2, 2)],
                             idxv.at[slot2], isem)

        pltpu.sync_copy(onesv, accd.at[idxv.at[lax.rem(j, 4), 1]], add=True)

    plsc.subcore_barrier()
    pltpu.sync_copy(accd.at[pl.ds(s * RPT, RPT)],
                    deg_out.at[pl.ds(c * N_PAD + s * RPT, RPT)])


def _sc_mesh():
    return plsc.VectorSubcoreMesh(core_axis_name="c", subcore_axis_name="s",
                                  num_cores=NC, num_subcores=NS)


@functools.lru_cache(maxsize=None)
def _make_agg():
    scratch = [
        pltpu.VMEM((4, 2, CHUNK), jnp.int32),
        pltpu.VMEM((CHUNK, D), jnp.float32),
        pltpu.VMEM((CHUNK, D), jnp.float32),
        pltpu.VMEM_SHARED((N_PAD, D), jnp.float32),
        pltpu.SemaphoreType.DMA, pltpu.SemaphoreType.DMA,
        pltpu.SemaphoreType.DMA,
    ]
    return pl.kernel(
        lambda *refs: _agg_body(refs),
        out_type=jax.ShapeDtypeStruct((NC * N_PAD, D), jnp.float32),
        mesh=_sc_mesh(),
        scratch_types=scratch,
        name="gin_agg",
    )


@functools.lru_cache(maxsize=None)
def _make_deg():
    scratch = [
        pltpu.VMEM((4, 2, CHUNK), jnp.int32),
        pltpu.VMEM((CHUNK, D), jnp.float32),
        pltpu.VMEM_SHARED((N_PAD, D), jnp.float32),
        pltpu.SemaphoreType.DMA,
    ]
    return pl.kernel(
        lambda *refs: _deg_body(refs),
        out_type=jax.ShapeDtypeStruct((NC * N_PAD, D), jnp.float32),
        mesh=_sc_mesh(),
        scratch_types=scratch,
        name="gin_deg",
    )


_USE_SC_AGG = True
_USE_SC_DEG = True


def _xla_agg(g, idx2, zeros):
    idx4 = idx2.reshape(NW, NCH, 2, CHUNK)
    src = idx4[:, :, 0, :].reshape(-1)
    dst = idx4[:, :, 1, :].reshape(-1)
    half = src.shape[0] // 2
    parts = []
    for c in range(2):
        sel = slice(c * half, (c + 1) * half)
        msg = jnp.take(g, src[sel], axis=0)
        parts.append(jax.ops.segment_sum(msg, dst[sel], num_segments=N_PAD))
    return jnp.concatenate(parts, axis=0)


def _xla_deg(idx2, zeros, ones8):
    idx4 = idx2.reshape(NW, NCH, 2, CHUNK)
    dst = idx4[:, :, 1, :].reshape(-1)
    half = dst.shape[0] // 2
    parts = []
    for c in range(2):
        sel = slice(c * half, (c + 1) * half)
        parts.append(jax.ops.segment_sum(
            jnp.ones((half, D), jnp.float32), dst[sel], num_segments=N_PAD))
    return jnp.concatenate(parts, axis=0)


def _agg(*args):
    if _USE_SC_AGG:
        return _make_agg()(*args)
    return _xla_agg(*args)


def _deg(*args):
    if _USE_SC_DEG:
        return _make_deg()(*args)
    return _xla_deg(*args)


_BLK = 1024
_GRID = N_PAD // _BLK


def _mm_body(x_ref, w_ref, o_ref):
    o_ref[...] = lax.dot_general(
        x_ref[...], w_ref[...], (((1,), (1,)), ((), ())),
        preferred_element_type=jnp.float32)


def _mm1(x, W):
    return pl.pallas_call(
        _mm_body,
        grid=(_GRID,),
        in_specs=[
            pl.BlockSpec((_BLK, D), lambda i: (i, 0)),
            pl.BlockSpec((D, D), lambda i: (0, 0)),
        ],
        out_specs=pl.BlockSpec((_BLK, D), lambda i: (i, 0)),
        out_shape=jax.ShapeDtypeStruct((N_PAD, D), jnp.float32),
    )(x, W)


def _combine_mm_body(g_ref, sa_ref, sb_ref, da_ref, db_ref, b_ref, w_ref,
                     o_ref):
    deg = da_ref[:, :1] + db_ref[:, :1]
    dinv = 1.0 / jnp.maximum(deg, 1.0)
    h = g_ref[...] + b_ref[...] + dinv * (sa_ref[...] + sb_ref[...])
    h = jnp.maximum(h, 0.0)
    o_ref[...] = lax.dot_general(
        h, w_ref[...], (((1,), (1,)), ((), ())),
        preferred_element_type=jnp.float32)


def _combine_mm(g, sa, sb, da, db, b, W):
    return pl.pallas_call(
        _combine_mm_body,
        grid=(_GRID,),
        in_specs=[
            pl.BlockSpec((_BLK, D), lambda i: (i, 0)),
            pl.BlockSpec((_BLK, D), lambda i: (i, 0)),
            pl.BlockSpec((_BLK, D), lambda i: (i, 0)),
            pl.BlockSpec((_BLK, D), lambda i: (i, 0)),
            pl.BlockSpec((_BLK, D), lambda i: (i, 0)),
            pl.BlockSpec((1, D), lambda i: (0, 0)),
            pl.BlockSpec((D, D), lambda i: (0, 0)),
        ],
        out_specs=pl.BlockSpec((_BLK, D), lambda i: (i, 0)),
        out_shape=jax.ShapeDtypeStruct((N_PAD, D), jnp.float32),
    )(g, sa, sb, da, db, b, W)


_OBLK = 1000


def _combine_out_body(g_ref, sa_ref, sb_ref, da_ref, db_ref, b_ref, o_ref):
    deg = da_ref[:, :1] + db_ref[:, :1]
    dinv = 1.0 / jnp.maximum(deg, 1.0)
    o_ref[...] = g_ref[...] + b_ref[...] + dinv * (sa_ref[...] + sb_ref[...])


def _combine_out(g, sa, sb, da, db, b):
    return pl.pallas_call(
        _combine_out_body,
        grid=(N // _OBLK,),
        in_specs=[
            pl.BlockSpec((_OBLK, D), lambda i: (i, 0)),
            pl.BlockSpec((_OBLK, D), lambda i: (i, 0)),
            pl.BlockSpec((_OBLK, D), lambda i: (i, 0)),
            pl.BlockSpec((_OBLK, D), lambda i: (i, 0)),
            pl.BlockSpec((_OBLK, D), lambda i: (i, 0)),
            pl.BlockSpec((1, D), lambda i: (0, 0)),
        ],
        out_specs=pl.BlockSpec((_OBLK, D), lambda i: (i, 0)),
        out_shape=jax.ShapeDtypeStruct((N, D), jnp.float32),
    )(g, sa, sb, da, db, b)


def kernel(x, edge_index, W1, b1, W2, b2, W3, b3):
    src = edge_index[0].astype(jnp.int32)
    dst = edge_index[1].astype(jnp.int32)
    pad = N + jnp.arange(E_PAD - E, dtype=jnp.int32) % (N_PAD - N)
    src3 = jnp.concatenate([src, pad]).reshape(NW, NCH, 1, CHUNK)
    dst3 = jnp.concatenate([dst, pad]).reshape(NW, NCH, 1, CHUNK)
    idx2 = jnp.concatenate([src3, dst3], axis=2).reshape(NW * NCH * 2, CHUNK)
    zeros = jnp.zeros((N_PAD, D), jnp.float32)
    ones8 = jnp.ones((8, D), jnp.float32)
    b1r = b1.reshape(1, D)
    b2r = b2.reshape(1, D)
    b3r = b3.reshape(1, D)

    g1 = _mm1(x, W1)
    deg = _deg(idx2, zeros, ones8)
    s1 = _agg(g1, idx2, zeros)
    da, db = deg[:N_PAD], deg[N_PAD:]
    g2 = _combine_mm(g1, s1[:N_PAD], s1[N_PAD:], da, db, b1r, W2)
    s2 = _agg(g2, idx2, zeros)
    g3 = _combine_mm(g2, s2[:N_PAD], s2[N_PAD:], da, db, b2r, W3)
    s3 = _agg(g3, idx2, zeros)
    return _combine_out(g3, s3[:N_PAD], s3[N_PAD:], da, db, b3r)


# speedup vs baseline: 9.5411x; 1.0485x over previous
import functools

import jax
import jax.numpy as jnp
from jax import lax
from jax.experimental import pallas as pl
from jax.experimental.pallas import tpu as pltpu
from jax.experimental.pallas import tpu_sc as plsc

N = 10000
E = 320000
D = 128
NC = 2
NS = 16
NW = NC * NS
CHUNK = 128
N_PAD = 10240
E_PAD = NW * 80 * CHUNK
NCH = E_PAD // (NW * CHUNK)
RPT = N_PAD // NS
PAD_IDX = N_PAD - 1


def _agg_body(refs):
    (g_hbm, idx_hbm, zeros_hbm,
     s_out,
     idxv, rows0, rows1, acc, sem0, sem1, isem) = refs

    c = lax.axis_index("c")
    s = lax.axis_index("s")
    wid = c * NS + s

    pltpu.sync_copy(zeros_hbm.at[pl.ds(s * RPT, RPT)],
                    acc.at[pl.ds(s * RPT, RPT)])
    plsc.subcore_barrier()

    rows = (rows0, rows1)
    sems = (sem0, sem1)

    ibase = wid * NCH * 2
    pltpu.sync_copy(idx_hbm.at[pl.ds(ibase, 2)], idxv.at[0])
    pltpu.async_copy(idx_hbm.at[pl.ds(ibase + 2, 2)], idxv.at[1], isem)
    pltpu.async_copy(g_hbm.at[idxv.at[0, 0]], rows0, sem0)

    @pl.loop(0, NCH, step=2)
    def _(jj):
        for b in (0, 1):
            j = jj + b
            pltpu.make_async_copy(g_hbm.at[pl.ds(0, CHUNK)], rows[b],
                                  sems[b]).wait()
            nxt = j + 1

            @pl.when(nxt < NCH)
            def _():
                slot = lax.rem(nxt, 4)
                pltpu.make_async_copy(idx_hbm.at[pl.ds(0, 2)], idxv.at[slot],
                                      isem).wait()
                pltpu.async_copy(g_hbm.at[idxv.at[slot, 0]], rows[1 - b],
                                 sems[1 - b])

            @pl.when(j + 2 < NCH)
            def _():
                slot2 = lax.rem(j + 2, 4)
                pltpu.async_copy(idx_hbm.at[pl.ds(ibase + (j + 2) * 2, 2)],
                                 idxv.at[slot2], isem)

            dslot = lax.rem(j, 4)
            pltpu.sync_copy(rows[b], acc.at[idxv.at[dslot, 1]], add=True)

    plsc.subcore_barrier()
    pltpu.sync_copy(acc.at[pl.ds(s * RPT, RPT)],
                    s_out.at[pl.ds(c * N_PAD + s * RPT, RPT)])


def _deg_body(refs):
    (idx_hbm, zeros_hbm, ones_hbm,
     deg_out,
     idxv, onesv, accd, isem) = refs

    c = lax.axis_index("c")
    s = lax.axis_index("s")
    wid = c * NS + s

    pltpu.sync_copy(zeros_hbm.at[pl.ds(s * RPT, RPT)],
                    accd.at[pl.ds(s * RPT, RPT)])
    for k in range(CHUNK // 8):
        pltpu.sync_copy(ones_hbm, onesv.at[pl.ds(k * 8, 8)])
    plsc.subcore_barrier()

    ibase = wid * NCH * 2
    pltpu.async_copy(idx_hbm.at[pl.ds(ibase, 2)], idxv.at[0], isem)
    pltpu.async_copy(idx_hbm.at[pl.ds(ibase + 2, 2)], idxv.at[1], isem)

    @pl.loop(0, NCH)
    def _(j):
        pltpu.make_async_copy(idx_hbm.at[pl.ds(0, 2)],
                              idxv.at[lax.rem(j, 4)], isem).wait()

        @pl.when(j + 2 < NCH)
        def _():
            slot2 = lax.rem(j + 2, 4)
            pltpu.async_copy(idx_hbm.at[pl.ds(ibase + (j + 2) * 2, 2)],
                             idxv.at[slot2], isem)

        pltpu.sync_copy(onesv, accd.at[idxv.at[lax.rem(j, 4), 1]], add=True)

    plsc.subcore_barrier()
    pltpu.sync_copy(accd.at[pl.ds(s * RPT, RPT)],
                    deg_out.at[pl.ds(c * N_PAD + s * RPT, RPT)])


def _sc_mesh():
    return plsc.VectorSubcoreMesh(core_axis_name="c", subcore_axis_name="s",
                                  num_cores=NC, num_subcores=NS)


@functools.lru_cache(maxsize=None)
def _make_agg():
    scratch = [
        pltpu.VMEM((4, 2, CHUNK), jnp.int32),
        pltpu.VMEM((CHUNK, D), jnp.float32),
        pltpu.VMEM((CHUNK, D), jnp.float32),
        pltpu.VMEM_SHARED((N_PAD, D), jnp.float32),
        pltpu.SemaphoreType.DMA, pltpu.SemaphoreType.DMA,
        pltpu.SemaphoreType.DMA,
    ]
    return pl.kernel(
        lambda *refs: _agg_body(refs),
        out_type=jax.ShapeDtypeStruct((NC * N_PAD, D), jnp.float32),
        mesh=_sc_mesh(),
        scratch_types=scratch,
        name="gin_agg",
    )


@functools.lru_cache(maxsize=None)
def _make_deg():
    scratch = [
        pltpu.VMEM((4, 2, CHUNK), jnp.int32),
        pltpu.VMEM((CHUNK, D), jnp.float32),
        pltpu.VMEM_SHARED((N_PAD, D), jnp.float32),
        pltpu.SemaphoreType.DMA,
    ]
    return pl.kernel(
        lambda *refs: _deg_body(refs),
        out_type=jax.ShapeDtypeStruct((NC * N_PAD, D), jnp.float32),
        mesh=_sc_mesh(),
        scratch_types=scratch,
        name="gin_deg",
    )


_USE_SC_AGG = True
_USE_SC_DEG = True


def _xla_agg(g, idx2, zeros):
    idx4 = idx2.reshape(NW, NCH, 2, CHUNK)
    src = idx4[:, :, 0, :].reshape(-1)
    dst = idx4[:, :, 1, :].reshape(-1)
    half = src.shape[0] // 2
    parts = []
    for c in range(2):
        sel = slice(c * half, (c + 1) * half)
        msg = jnp.take(g, src[sel], axis=0)
        parts.append(jax.ops.segment_sum(msg, dst[sel], num_segments=N_PAD))
    return jnp.concatenate(parts, axis=0)


def _xla_deg(idx2, zeros, ones8):
    idx4 = idx2.reshape(NW, NCH, 2, CHUNK)
    dst = idx4[:, :, 1, :].reshape(-1)
    half = dst.shape[0] // 2
    parts = []
    for c in range(2):
        sel = slice(c * half, (c + 1) * half)
        parts.append(jax.ops.segment_sum(
            jnp.ones((half, D), jnp.float32), dst[sel], num_segments=N_PAD))
    return jnp.concatenate(parts, axis=0)


def _agg(*args):
    if _USE_SC_AGG:
        return _make_agg()(*args)
    return _xla_agg(*args)


def _deg(*args):
    if _USE_SC_DEG:
        return _make_deg()(*args)
    return _xla_deg(*args)


_BLK = 1024
_GRID = N_PAD // _BLK


def _mm_body(x_ref, w_ref, o_ref):
    o_ref[...] = lax.dot_general(
        x_ref[...], w_ref[...], (((1,), (1,)), ((), ())),
        preferred_element_type=jnp.float32)


def _mm1(x, W):
    return pl.pallas_call(
        _mm_body,
        grid=(_GRID,),
        in_specs=[
            pl.BlockSpec((_BLK, D), lambda i: (i, 0)),
            pl.BlockSpec((D, D), lambda i: (0, 0)),
        ],
        out_specs=pl.BlockSpec((_BLK, D), lambda i: (i, 0)),
        out_shape=jax.ShapeDtypeStruct((N_PAD, D), jnp.float32),
    )(x, W)


def _combine_mm_body(g_ref, sa_ref, sb_ref, da_ref, db_ref, b_ref, w_ref,
                     o_ref):
    deg = da_ref[:, :1] + db_ref[:, :1]
    dinv = 1.0 / jnp.maximum(deg, 1.0)
    h = g_ref[...] + b_ref[...] + dinv * (sa_ref[...] + sb_ref[...])
    h = jnp.maximum(h, 0.0)
    o_ref[...] = lax.dot_general(
        h, w_ref[...], (((1,), (1,)), ((), ())),
        preferred_element_type=jnp.float32)


_HGRID = N_PAD // _BLK


def _combine_mm(g, s, deg, b, W):
    return pl.pallas_call(
        _combine_mm_body,
        grid=(_GRID,),
        in_specs=[
            pl.BlockSpec((_BLK, D), lambda i: (i, 0)),
            pl.BlockSpec((_BLK, D), lambda i: (i, 0)),
            pl.BlockSpec((_BLK, D), lambda i: (i + _HGRID, 0)),
            pl.BlockSpec((_BLK, D), lambda i: (i, 0)),
            pl.BlockSpec((_BLK, D), lambda i: (i + _HGRID, 0)),
            pl.BlockSpec((1, D), lambda i: (0, 0)),
            pl.BlockSpec((D, D), lambda i: (0, 0)),
        ],
        out_specs=pl.BlockSpec((_BLK, D), lambda i: (i, 0)),
        out_shape=jax.ShapeDtypeStruct((N_PAD, D), jnp.float32),
    )(g, s, s, deg, deg, b, W)


def _combine_out_body(g_ref, sa_ref, sb_ref, da_ref, db_ref, b_ref, o_ref):
    deg = da_ref[:, :1] + db_ref[:, :1]
    dinv = 1.0 / jnp.maximum(deg, 1.0)
    o_ref[...] = g_ref[...] + b_ref[...] + dinv * (sa_ref[...] + sb_ref[...])


def _combine_out(g, s, deg, b):
    return pl.pallas_call(
        _combine_out_body,
        grid=(_GRID,),
        in_specs=[
            pl.BlockSpec((_BLK, D), lambda i: (i, 0)),
            pl.BlockSpec((_BLK, D), lambda i: (i, 0)),
            pl.BlockSpec((_BLK, D), lambda i: (i + _HGRID, 0)),
            pl.BlockSpec((_BLK, D), lambda i: (i, 0)),
            pl.BlockSpec((_BLK, D), lambda i: (i + _HGRID, 0)),
            pl.BlockSpec((1, D), lambda i: (0, 0)),
        ],
        out_specs=pl.BlockSpec((_BLK, D), lambda i: (i, 0)),
        out_shape=jax.ShapeDtypeStruct((N, D), jnp.float32),
    )(g, s, s, deg, deg, b)


def kernel(x, edge_index, W1, b1, W2, b2, W3, b3):
    src = edge_index[0].astype(jnp.int32)
    dst = edge_index[1].astype(jnp.int32)
    pad = N + jnp.arange(E_PAD - E, dtype=jnp.int32) % (N_PAD - N)
    src3 = jnp.concatenate([src, pad]).reshape(NW, NCH, 1, CHUNK)
    dst3 = jnp.concatenate([dst, pad]).reshape(NW, NCH, 1, CHUNK)
    idx2 = jnp.concatenate([src3, dst3], axis=2).reshape(NW * NCH * 2, CHUNK)
    zeros = jnp.zeros((N_PAD, D), jnp.float32)
    ones8 = jnp.ones((8, D), jnp.float32)
    b1r = b1.reshape(1, D)
    b2r = b2.reshape(1, D)
    b3r = b3.reshape(1, D)

    g1 = _mm1(x, W1)
    deg = _deg(idx2, zeros, ones8)
    s1 = _agg(g1, idx2, zeros)
    g2 = _combine_mm(g1, s1, deg, b1r, W2)
    s2 = _agg(g2, idx2, zeros)
    g3 = _combine_mm(g2, s2, deg, b2r, W3)
    s3 = _agg(g3, idx2, zeros)
    return _combine_out(g3, s3, deg, b3r)
